# Initial kernel scaffold; baseline (speedup 1.0000x reference)
import functools

import jax
import jax.numpy as jnp
from jax import lax
from jax.experimental import pallas as pl
from jax.experimental.pallas import tpu as pltpu
from jax.experimental.pallas import tpu_sc as plsc

_NC = 2
_NS = 16
_NW = _NC * _NS
_LANES = 16


def _row_segments(L):
    segs = []
    off = 0
    while off < L:
        n = min(128, L - off)
        segs.append((off, n))
        off += n
    return segs


def _pool_sums(x_flat, table, B, L, E):
    R = B // _NW
    CB = 8
    NCHUNK = R // CB
    IDX = CB * L
    EG = E // _LANES
    U = 4
    segs = _row_segments(L)

    mesh = plsc.VectorSubcoreMesh(core_axis_name="c", subcore_axis_name="s")

    @functools.partial(
        pl.kernel,
        out_type=jax.ShapeDtypeStruct((B, E), jnp.float32),
        mesh=mesh,
        scratch_types=[
            pltpu.VMEM((IDX,), jnp.int32),
            pltpu.VMEM((IDX, E), jnp.float32),
            pltpu.VMEM((R, E), jnp.float32),
            pltpu.SemaphoreType.DMA,
        ],
    )
    def pool(x_hbm, tab_hbm, out_hbm, idx_v, rows_v, acc_v, sem):
        w = lax.axis_index("s") * _NC + lax.axis_index("c")
        row0 = w * R

        @pl.loop(0, NCHUNK)
        def _chunk(k):
            fbase = (row0 + k * CB) * L
            pltpu.sync_copy(x_hbm.at[pl.ds(fbase, IDX)], idx_v)
            descs = []
            for b in range(CB):
                for (o, n) in segs:
                    src = tab_hbm.at[idx_v.at[pl.ds(b * L + o, n)]]
                    dst = rows_v.at[pl.ds(b * L + o, n)]
                    descs.append(pltpu.async_copy(src, dst, sem))
            for d in descs:
                d.wait()
            for b in range(CB):
                base = b * L

                def body(j, accs, base=base):
                    r = base + j * U
                    out = list(accs)
                    for u in range(U):
                        for g in range(EG):
                            out[g] = out[g] + rows_v[r + u, pl.ds(g * _LANES, _LANES)]
                    return tuple(out)

                accs = lax.fori_loop(
                    0, L // U, body,
                    tuple(jnp.zeros((_LANES,), jnp.float32) for _ in range(EG)))
                row = k * CB + b
                for g in range(EG):
                    acc_v[row, pl.ds(g * _LANES, _LANES)] = accs[g]

        pltpu.sync_copy(acc_v, out_hbm.at[pl.ds(row0, R)])

    return pool(x_flat, table)


def _mlp(pooled, w1t, b1r, w2t, b2r, inv_l):
    B = pooled.shape[0]
    OUT = w2t.shape[1]

    def body(s_ref, w1_ref, b1_ref, w2_ref, b2_ref, o_ref):
        h = s_ref[...] * inv_l
        h = jnp.dot(h, w1_ref[...], preferred_element_type=jnp.float32) + b1_ref[...]
        h = jnp.maximum(h, 0.0)
        o = jnp.dot(h, w2_ref[...], preferred_element_type=jnp.float32) + b2_ref[...]
        o_ref[...] = 1.0 / (1.0 + jnp.exp(-o))

    return pl.pallas_call(
        body,
        out_shape=jax.ShapeDtypeStruct((B, OUT), jnp.float32),
    )(pooled, w1t, b1r, w2t, b2r)


def kernel(x, table, W1, b1, W2, b2):
    B, L = x.shape
    _, E = table.shape
    HID = W1.shape[0]
    OUT = W2.shape[0]
    assert B % _NW == 0 and L % 8 == 0 and E % _LANES == 0

    x_flat = x.reshape(B * L).astype(jnp.int32)
    pooled = _pool_sums(x_flat, table, B, L, E)
    return _mlp(
        pooled,
        W1.T,
        b1.reshape(1, HID),
        W2.T,
        b2.reshape(1, OUT),
        1.0 / L,
    )


# speedup vs baseline: 2.2666x; 2.2666x over previous
import functools

import jax
import jax.numpy as jnp
from jax import lax
from jax.experimental import pallas as pl
from jax.experimental.pallas import tpu as pltpu
from jax.experimental.pallas import tpu_sc as plsc

_NC = 2
_NS = 16
_NW = _NC * _NS
_LANES = 16


def _row_segments(L):
    segs = []
    off = 0
    while off < L:
        n = min(128, L - off)
        segs.append((off, n))
        off += n
    return segs


def _pool_sums(x_flat, table, B, L, E):
    R = B // _NW
    CB = 8
    NCHUNK = R // CB
    IDX = CB * L
    EG = E // _LANES
    U = 4
    segs = _row_segments(L)

    mesh = plsc.VectorSubcoreMesh(core_axis_name="c", subcore_axis_name="s")

    @functools.partial(
        pl.kernel,
        out_type=jax.ShapeDtypeStruct((B, E), jnp.float32),
        mesh=mesh,
        compiler_params=pltpu.CompilerParams(use_tc_tiling_on_sc=False),
        scratch_types=[
            pltpu.VMEM((IDX,), jnp.int32),
            pltpu.VMEM((IDX, E), jnp.float32),
            pltpu.VMEM((R, E), jnp.float32),
            pltpu.SemaphoreType.DMA,
        ],
    )
    def pool(x_hbm, tab_hbm, out_hbm, idx_v, rows_v, acc_v, sem):
        w = lax.axis_index("s") * _NC + lax.axis_index("c")
        row0 = w * R

        @pl.loop(0, NCHUNK)
        def _chunk(k):
            fbase = (row0 + k * CB) * L
            pltpu.sync_copy(x_hbm.at[pl.ds(fbase, IDX)], idx_v)
            descs = []
            for b in range(CB):
                for (o, n) in segs:
                    src = tab_hbm.at[idx_v.at[pl.ds(b * L + o, n)]]
                    dst = rows_v.at[pl.ds(b * L + o, n)]
                    descs.append(pltpu.async_copy(src, dst, sem))
            for d in descs:
                d.wait()
            for b in range(CB):
                base = b * L

                def body(j, accs, base=base):
                    r = base + j * U
                    out = list(accs)
                    for u in range(U):
                        for g in range(EG):
                            out[g] = out[g] + rows_v[r + u, pl.ds(g * _LANES, _LANES)]
                    return tuple(out)

                accs = lax.fori_loop(
                    0, L // U, body,
                    tuple(jnp.zeros((_LANES,), jnp.float32) for _ in range(EG)))
                row = k * CB + b
                for g in range(EG):
                    acc_v[row, pl.ds(g * _LANES, _LANES)] = accs[g]

        pltpu.sync_copy(acc_v, out_hbm.at[pl.ds(row0, R)])

    return pool(x_flat, table)


def _mlp(pooled, w1t, b1r, w2t, b2r, inv_l):
    B = pooled.shape[0]
    OUT = w2t.shape[1]

    def body(s_ref, w1_ref, b1_ref, w2_ref, b2_ref, o_ref):
        h = s_ref[...] * inv_l
        h = jnp.dot(h, w1_ref[...], preferred_element_type=jnp.float32) + b1_ref[...]
        h = jnp.maximum(h, 0.0)
        o = jnp.dot(h, w2_ref[...], preferred_element_type=jnp.float32) + b2_ref[...]
        o_ref[...] = 1.0 / (1.0 + jnp.exp(-o))

    return pl.pallas_call(
        body,
        out_shape=jax.ShapeDtypeStruct((B, OUT), jnp.float32),
    )(pooled, w1t, b1r, w2t, b2r)


def kernel(x, table, W1, b1, W2, b2):
    B, L = x.shape
    _, E = table.shape
    HID = W1.shape[0]
    OUT = W2.shape[0]
    assert B % _NW == 0 and L % 8 == 0 and E % _LANES == 0

    x_flat = x.reshape(B * L).astype(jnp.int32)
    pooled = _pool_sums(x_flat, table, B, L, E)
    return _mlp(
        pooled,
        W1.T,
        b1.reshape(1, HID),
        W2.T,
        b2.reshape(1, OUT),
        1.0 / L,
    )


# speedup vs baseline: 2.4086x; 1.0626x over previous
import functools

import jax
import jax.numpy as jnp
from jax import lax
from jax.experimental import pallas as pl
from jax.experimental.pallas import tpu as pltpu
from jax.experimental.pallas import tpu_sc as plsc

_NC = 2
_NS = 16
_NW = _NC * _NS
_LANES = 16


def _row_segments(L):
    segs = []
    off = 0
    while off < L:
        n = min(128, L - off)
        segs.append((off, n))
        off += n
    return segs


def _pool_sums(x, table, B, L, E):
    R = B // _NW
    CB = 4
    NCHUNK = R // CB
    EG = E // _LANES
    U = 4
    segs = _row_segments(L)
    assert NCHUNK % 2 == 0 and L % U == 0

    mesh = plsc.VectorSubcoreMesh(core_axis_name="c", subcore_axis_name="s")

    @functools.partial(
        pl.kernel,
        out_type=jax.ShapeDtypeStruct((B, E), jnp.float32),
        mesh=mesh,
        compiler_params=pltpu.CompilerParams(use_tc_tiling_on_sc=False),
        scratch_types=[
            pltpu.VMEM((R, L), jnp.int32),
            pltpu.VMEM((CB * L, E), jnp.float32),
            pltpu.VMEM((CB * L, E), jnp.float32),
            pltpu.VMEM((R, E), jnp.float32),
            pltpu.SemaphoreType.DMA,
            pltpu.SemaphoreType.DMA,
        ],
    )
    def pool(x_hbm, tab_hbm, out_hbm, idx_v, rows_a, rows_b, acc_v, sem_a, sem_b):
        w = lax.axis_index("s") * _NC + lax.axis_index("c")
        row0 = w * R

        def copies(c, buf, sem):
            out = []
            for b in range(CB):
                for (o, n) in segs:
                    src = tab_hbm.at[idx_v.at[c * CB + b, pl.ds(o, n)]]
                    dst = buf.at[pl.ds(b * L + o, n)]
                    out.append(pltpu.make_async_copy(src, dst, sem))
            return out

        def fire(c, buf, sem):
            for d in copies(c, buf, sem):
                d.start()

        def drain(c, buf, sem):
            for d in copies(c, buf, sem):
                d.wait()

        def compute(c, buf):
            for b in range(CB):
                base = b * L

                def body(j, accs, base=base):
                    r = base + j * U
                    out = list(accs)
                    for g in range(EG):
                        s = pl.ds(g * _LANES, _LANES)
                        out[2 * g] = out[2 * g] + buf[r, s] + buf[r + 1, s]
                        out[2 * g + 1] = out[2 * g + 1] + buf[r + 2, s] + buf[r + 3, s]
                    return tuple(out)

                accs = lax.fori_loop(
                    0, L // U, body,
                    tuple(jnp.zeros((_LANES,), jnp.float32) for _ in range(2 * EG)))
                row = c * CB + b
                for g in range(EG):
                    acc_v[row, pl.ds(g * _LANES, _LANES)] = accs[2 * g] + accs[2 * g + 1]

        pltpu.sync_copy(x_hbm.at[pl.ds(row0, R)], idx_v)
        fire(0, rows_a, sem_a)

        @pl.loop(0, NCHUNK - 2, step=2)
        def _pair(c0):
            fire(c0 + 1, rows_b, sem_b)
            drain(c0, rows_a, sem_a)
            compute(c0, rows_a)
            fire(c0 + 2, rows_a, sem_a)
            drain(c0 + 1, rows_b, sem_b)
            compute(c0 + 1, rows_b)

        c0 = NCHUNK - 2
        fire(c0 + 1, rows_b, sem_b)
        drain(c0, rows_a, sem_a)
        compute(c0, rows_a)
        drain(c0 + 1, rows_b, sem_b)
        compute(c0 + 1, rows_b)

        pltpu.sync_copy(acc_v, out_hbm.at[pl.ds(row0, R)])

    return pool(x, table)


def _mlp(pooled, w1t, b1r, w2t, b2r, inv_l):
    B = pooled.shape[0]
    OUT = w2t.shape[1]

    def body(s_ref, w1_ref, b1_ref, w2_ref, b2_ref, o_ref):
        h = s_ref[...] * inv_l
        h = jnp.dot(h, w1_ref[...], preferred_element_type=jnp.float32) + b1_ref[...]
        h = jnp.maximum(h, 0.0)
        o = jnp.dot(h, w2_ref[...], preferred_element_type=jnp.float32) + b2_ref[...]
        o_ref[...] = 1.0 / (1.0 + jnp.exp(-o))

    return pl.pallas_call(
        body,
        out_shape=jax.ShapeDtypeStruct((B, OUT), jnp.float32),
    )(pooled, w1t, b1r, w2t, b2r)


def kernel(x, table, W1, b1, W2, b2):
    B, L = x.shape
    _, E = table.shape
    HID = W1.shape[0]
    OUT = W2.shape[0]
    assert B % _NW == 0 and L % 8 == 0 and E % _LANES == 0

    pooled = _pool_sums(x.astype(jnp.int32), table, B, L, E)
    return _mlp(
        pooled,
        W1.T,
        b1.reshape(1, HID),
        W2.T,
        b2.reshape(1, OUT),
        1.0 / L,
    )
